# baseline (speedup 1.0000x reference)
import functools

import jax
import jax.numpy as jnp
from jax import lax
from jax.experimental import pallas as pl
from jax.experimental.pallas import tpu as pltpu
from jax.experimental.pallas import tpu_sc as plsc

_N_BINS = 10
_ROWS = 2048
_LANES = 16
_NTILES = 32
_CHUNK = 16384


def _dense_body(bounds_ref, logits_ref, labels_ref, conf_ref, acc_ref, bin_ref):
    x = logits_ref[...]
    rows, ncls = x.shape
    m = jnp.max(x, axis=1, keepdims=True)
    s = jnp.sum(jnp.exp(x - m), axis=1)
    conf = 1.0 / s
    col = lax.broadcasted_iota(jnp.int32, (rows, ncls), 1)
    pred = jnp.min(jnp.where(x == m, col, ncls), axis=1)
    lab = labels_ref[0, 0, :]
    accf = (pred == lab).astype(jnp.float32)
    binv = jnp.zeros((rows,), jnp.int32)
    for j in range(1, _N_BINS):
        binv = binv + (conf > bounds_ref[j]).astype(jnp.int32)
    conf_ref[0, 0, :] = conf
    acc_ref[0, 0, :] = accf
    bin_ref[0, 0, :] = binv


def _dense_stage(bounds, logits, labels3):
    n, ncls = logits.shape
    nb = n // _ROWS
    blk3 = pl.BlockSpec((1, 1, _ROWS), lambda i: (i, 0, 0))
    return pl.pallas_call(
        _dense_body,
        grid=(nb,),
        in_specs=[
            pl.BlockSpec(memory_space=pltpu.SMEM),
            pl.BlockSpec((_ROWS, ncls), lambda i: (i, 0)),
            blk3,
        ],
        out_specs=[blk3, blk3, blk3],
        out_shape=[
            jax.ShapeDtypeStruct((nb, 1, _ROWS), jnp.float32),
            jax.ShapeDtypeStruct((nb, 1, _ROWS), jnp.float32),
            jax.ShapeDtypeStruct((nb, 1, _ROWS), jnp.int32),
        ],
    )(bounds, logits, labels3)


def _make_hist_kernel(n):
    per_tile = n // _NTILES
    n_chunks = per_tile // _CHUNK
    mesh = plsc.VectorSubcoreMesh(core_axis_name="c", subcore_axis_name="s")

    @functools.partial(
        pl.kernel,
        out_type=jax.ShapeDtypeStruct((_NTILES, 3, _LANES), jnp.float32),
        mesh=mesh,
        compiler_params=pltpu.CompilerParams(needs_layout_passes=False),
        scratch_types=[
            pltpu.VMEM((_CHUNK,), jnp.int32),
            pltpu.VMEM((_CHUNK,), jnp.float32),
            pltpu.VMEM((_CHUNK,), jnp.float32),
            pltpu.VMEM((_LANES * _LANES,), jnp.float32),
            pltpu.VMEM((_LANES * _LANES,), jnp.float32),
            pltpu.VMEM((_LANES * _LANES,), jnp.float32),
            pltpu.VMEM((3, _LANES), jnp.float32),
        ],
    )
    def hist(bin_hbm, conf_hbm, acc_hbm, out_hbm,
             bin_v, conf_v, acc_v, cnt_a, csum_a, asum_a, res_v):
        cid = lax.axis_index("c")
        sid = lax.axis_index("s")
        wid = sid * 2 + cid
        zero16 = jnp.zeros((_LANES,), jnp.float32)
        for r in range(_LANES):
            sl = pl.ds(r * _LANES, _LANES)
            cnt_a[sl] = zero16
            csum_a[sl] = zero16
            asum_a[sl] = zero16
        lanes = lax.iota(jnp.int32, _LANES)
        lane_off = lanes * _LANES
        ones = jnp.ones((_LANES,), jnp.float32)
        base0 = wid * per_tile
        for c in range(n_chunks):
            base = base0 + c * _CHUNK
            pltpu.sync_copy(bin_hbm.at[pl.ds(base, _CHUNK)], bin_v)
            pltpu.sync_copy(conf_hbm.at[pl.ds(base, _CHUNK)], conf_v)
            pltpu.sync_copy(acc_hbm.at[pl.ds(base, _CHUNK)], acc_v)

            def body(i, _):
                b = bin_v[pl.ds(i * _LANES, _LANES)]
                v = conf_v[pl.ds(i * _LANES, _LANES)]
                a = acc_v[pl.ds(i * _LANES, _LANES)]
                addr = lane_off + b
                plsc.addupdate_scatter(cnt_a, [addr], ones)
                plsc.addupdate_scatter(csum_a, [addr], v)
                plsc.addupdate_scatter(asum_a, [addr], a)
                return 0

            lax.fori_loop(0, _CHUNK // _LANES, body, 0)
        cnt_t = zero16
        csum_t = zero16
        asum_t = zero16
        for r in range(_LANES):
            sl = pl.ds(r * _LANES, _LANES)
            cnt_t = cnt_t + cnt_a[sl]
            csum_t = csum_t + csum_a[sl]
            asum_t = asum_t + asum_a[sl]
        res_v[0] = cnt_t
        res_v[1] = csum_t
        res_v[2] = asum_t
        pltpu.sync_copy(res_v, out_hbm.at[wid])

    return hist


def kernel(logits, labels):
    n, _ = logits.shape
    bounds = jnp.linspace(0.0, 1.0, _N_BINS + 1).astype(jnp.float32)
    labels3 = labels.reshape(n // _ROWS, 1, _ROWS)
    conf3, acc3, bin3 = _dense_stage(bounds, logits, labels3)
    partials = _make_hist_kernel(n)(
        bin3.reshape(n), conf3.reshape(n), acc3.reshape(n))
    stats = jnp.sum(partials, axis=0)
    cnt = stats[0, :_N_BINS]
    csum = stats[1, :_N_BINS]
    asum = stats[2, :_N_BINS]
    nf = jnp.float32(n)
    safe = jnp.maximum(cnt, 1.0)
    contrib = jnp.abs(csum / safe - asum / safe) * (cnt / nf)
    ece = jnp.sum(jnp.where(cnt > 0, contrib, 0.0), keepdims=True)
    acc = jnp.sum(stats[2]) / nf
    return ece, acc


# speedup vs baseline: 2.9715x; 2.9715x over previous
import functools

import jax
import jax.numpy as jnp
from jax import lax
from jax.experimental import pallas as pl
from jax.experimental.pallas import tpu as pltpu
from jax.experimental.pallas import tpu_sc as plsc

_N_BINS = 10
_ROWS = 2048
_LANES = 16
_NTILES = 32
_NCELLS = 32
_CHUNK = 16384


def _dense_body(bounds_ref, logits_ref, labels_ref, conf_ref, idx_ref):
    x = logits_ref[...]
    rows, ncls = x.shape
    xt = x.T
    m = jnp.max(xt, axis=0)
    s = jnp.sum(jnp.exp(xt - m[None, :]), axis=0)
    conf = 1.0 / s
    row = lax.broadcasted_iota(jnp.int32, (ncls, rows), 0)
    pred = jnp.min(jnp.where(xt == m[None, :], row, ncls), axis=0)
    lab = labels_ref[0, 0, :]
    acci = (pred == lab).astype(jnp.int32)
    binv = jnp.zeros((rows,), jnp.int32)
    for j in range(1, _N_BINS):
        binv = binv + (conf > bounds_ref[j]).astype(jnp.int32)
    conf_ref[0, 0, :] = conf
    idx_ref[0, 0, :] = binv * 2 + acci


def _dense_stage(bounds, logits, labels3):
    n, ncls = logits.shape
    nb = n // _ROWS
    blk3 = pl.BlockSpec((1, 1, _ROWS), lambda i: (i, 0, 0))
    return pl.pallas_call(
        _dense_body,
        grid=(nb,),
        in_specs=[
            pl.BlockSpec(memory_space=pltpu.SMEM),
            pl.BlockSpec((_ROWS, ncls), lambda i: (i, 0)),
            blk3,
        ],
        out_specs=[blk3, blk3],
        out_shape=[
            jax.ShapeDtypeStruct((nb, 1, _ROWS), jnp.float32),
            jax.ShapeDtypeStruct((nb, 1, _ROWS), jnp.int32),
        ],
    )(bounds, logits, labels3)


def _make_hist_kernel(n):
    per_tile = n // _NTILES
    n_chunks = per_tile // _CHUNK
    nacc = _LANES * _NCELLS
    mesh = plsc.VectorSubcoreMesh(core_axis_name="c", subcore_axis_name="s")

    @functools.partial(
        pl.kernel,
        out_type=jax.ShapeDtypeStruct((_NTILES, 4, _LANES), jnp.float32),
        mesh=mesh,
        compiler_params=pltpu.CompilerParams(needs_layout_passes=False),
        scratch_types=[
            pltpu.VMEM((_CHUNK,), jnp.int32),
            pltpu.VMEM((_CHUNK,), jnp.float32),
            pltpu.VMEM((nacc,), jnp.float32),
            pltpu.VMEM((nacc,), jnp.float32),
            pltpu.VMEM((4, _LANES), jnp.float32),
        ],
    )
    def hist(idx_hbm, conf_hbm, out_hbm,
             idx_v, conf_v, cnt_a, csum_a, res_v):
        cid = lax.axis_index("c")
        sid = lax.axis_index("s")
        wid = sid * 2 + cid
        zero16 = jnp.zeros((_LANES,), jnp.float32)
        for r in range(nacc // _LANES):
            sl = pl.ds(r * _LANES, _LANES)
            cnt_a[sl] = zero16
            csum_a[sl] = zero16
        lanes = lax.iota(jnp.int32, _LANES)
        lane_off = lanes * _NCELLS
        ones = jnp.ones((_LANES,), jnp.float32)
        base0 = wid * per_tile
        for c in range(n_chunks):
            base = base0 + c * _CHUNK
            pltpu.sync_copy(idx_hbm.at[pl.ds(base, _CHUNK)], idx_v)
            pltpu.sync_copy(conf_hbm.at[pl.ds(base, _CHUNK)], conf_v)

            def body(i, _):
                b = idx_v[pl.ds(i * _LANES, _LANES)]
                v = conf_v[pl.ds(i * _LANES, _LANES)]
                addr = lane_off + b
                plsc.addupdate_scatter(cnt_a, [addr], ones)
                plsc.addupdate_scatter(csum_a, [addr], v)
                return 0

            lax.fori_loop(0, _CHUNK // _LANES, body, 0)
        tot = [zero16, zero16, zero16, zero16]
        for r in range(_LANES):
            for h in range(2):
                sl = pl.ds(r * _NCELLS + h * _LANES, _LANES)
                tot[h] = tot[h] + cnt_a[sl]
                tot[2 + h] = tot[2 + h] + csum_a[sl]
        for k in range(4):
            res_v[k] = tot[k]
        pltpu.sync_copy(res_v, out_hbm.at[wid])

    return hist


def kernel(logits, labels):
    n, _ = logits.shape
    bounds = jnp.linspace(0.0, 1.0, _N_BINS + 1).astype(jnp.float32)
    labels3 = labels.reshape(n // _ROWS, 1, _ROWS)
    conf3, idx3 = _dense_stage(bounds, logits, labels3)
    partials = _make_hist_kernel(n)(idx3.reshape(n), conf3.reshape(n))
    stats = jnp.sum(partials, axis=0)
    cnt_c = jnp.concatenate([stats[0], stats[1]]).reshape(_LANES, 2)
    csum_c = jnp.concatenate([stats[2], stats[3]]).reshape(_LANES, 2)
    cnt = cnt_c[:_N_BINS, 0] + cnt_c[:_N_BINS, 1]
    asum = cnt_c[:_N_BINS, 1]
    csum = csum_c[:_N_BINS, 0] + csum_c[:_N_BINS, 1]
    nf = jnp.float32(n)
    safe = jnp.maximum(cnt, 1.0)
    contrib = jnp.abs(csum / safe - asum / safe) * (cnt / nf)
    ece = jnp.sum(jnp.where(cnt > 0, contrib, 0.0), keepdims=True)
    acc = jnp.sum(cnt_c[:, 1]) / nf
    return ece, acc


# speedup vs baseline: 3.4605x; 1.1646x over previous
import functools

import jax
import jax.numpy as jnp
from jax import lax
from jax.experimental import pallas as pl
from jax.experimental.pallas import tpu as pltpu
from jax.experimental.pallas import tpu_sc as plsc

_N_BINS = 10
_ROWS = 4096
_LANES = 16
_NTILES = 32
_NCELLS = 32
_CHUNK = 16384


def _dense_body(bounds_ref, logits_ref, labels_ref, conf_ref, idx_ref):
    x = logits_ref[...]
    rows, ncls = x.shape
    xt = x.T
    m = jnp.max(xt, axis=0)
    s = jnp.sum(jnp.exp(xt - m[None, :]), axis=0)
    conf = 1.0 / s
    row = lax.broadcasted_iota(jnp.int32, (ncls, rows), 0)
    pred = jnp.min(jnp.where(xt == m[None, :], row, ncls), axis=0)
    lab = labels_ref[0, 0, :]
    acci = (pred == lab).astype(jnp.int32)
    binv = jnp.zeros((rows,), jnp.int32)
    for j in range(1, _N_BINS):
        binv = binv + (conf > bounds_ref[j]).astype(jnp.int32)
    conf_ref[0, 0, :] = conf
    idx_ref[0, 0, :] = binv * 2 + acci


def _dense_stage(bounds, logits, labels3):
    n, ncls = logits.shape
    nb = n // _ROWS
    blk3 = pl.BlockSpec((1, 1, _ROWS), lambda i: (i, 0, 0))
    return pl.pallas_call(
        _dense_body,
        grid=(nb,),
        in_specs=[
            pl.BlockSpec(memory_space=pltpu.SMEM),
            pl.BlockSpec((_ROWS, ncls), lambda i: (i, 0)),
            blk3,
        ],
        out_specs=[blk3, blk3],
        out_shape=[
            jax.ShapeDtypeStruct((nb, 1, _ROWS), jnp.float32),
            jax.ShapeDtypeStruct((nb, 1, _ROWS), jnp.int32),
        ],
    )(bounds, logits, labels3)


def _make_hist_kernel(n):
    per_tile = n // _NTILES
    n_chunks = per_tile // _CHUNK
    nacc = _LANES * _NCELLS
    mesh = plsc.VectorSubcoreMesh(core_axis_name="c", subcore_axis_name="s")

    @functools.partial(
        pl.kernel,
        out_type=jax.ShapeDtypeStruct((_NTILES, 4, _LANES), jnp.float32),
        mesh=mesh,
        compiler_params=pltpu.CompilerParams(needs_layout_passes=False),
        scratch_types=[
            pltpu.VMEM((_CHUNK,), jnp.int32),
            pltpu.VMEM((_CHUNK,), jnp.float32),
            pltpu.VMEM((nacc,), jnp.float32),
            pltpu.VMEM((nacc,), jnp.float32),
            pltpu.VMEM((4, _LANES), jnp.float32),
        ],
    )
    def hist(idx_hbm, conf_hbm, out_hbm,
             idx_v, conf_v, cnt_a, csum_a, res_v):
        cid = lax.axis_index("c")
        sid = lax.axis_index("s")
        wid = sid * 2 + cid
        zero16 = jnp.zeros((_LANES,), jnp.float32)
        for r in range(nacc // _LANES):
            sl = pl.ds(r * _LANES, _LANES)
            cnt_a[sl] = zero16
            csum_a[sl] = zero16
        lanes = lax.iota(jnp.int32, _LANES)
        lane_off = lanes * _NCELLS
        ones = jnp.ones((_LANES,), jnp.float32)
        base0 = wid * per_tile
        for c in range(n_chunks):
            base = base0 + c * _CHUNK
            pltpu.sync_copy(idx_hbm.at[pl.ds(base, _CHUNK)], idx_v)
            pltpu.sync_copy(conf_hbm.at[pl.ds(base, _CHUNK)], conf_v)

            def body(i, _):
                b = idx_v[pl.ds(i * _LANES, _LANES)]
                v = conf_v[pl.ds(i * _LANES, _LANES)]
                addr = lane_off + b
                plsc.addupdate_scatter(cnt_a, [addr], ones)
                plsc.addupdate_scatter(csum_a, [addr], v)
                return 0

            lax.fori_loop(0, _CHUNK // _LANES, body, 0)
        tot = [zero16, zero16, zero16, zero16]
        for r in range(_LANES):
            for h in range(2):
                sl = pl.ds(r * _NCELLS + h * _LANES, _LANES)
                tot[h] = tot[h] + cnt_a[sl]
                tot[2 + h] = tot[2 + h] + csum_a[sl]
        for k in range(4):
            res_v[k] = tot[k]
        pltpu.sync_copy(res_v, out_hbm.at[wid])

    return hist


def kernel(logits, labels):
    n, _ = logits.shape
    bounds = jnp.linspace(0.0, 1.0, _N_BINS + 1).astype(jnp.float32)
    labels3 = labels.reshape(n // _ROWS, 1, _ROWS)
    conf3, idx3 = _dense_stage(bounds, logits, labels3)
    partials = _make_hist_kernel(n)(idx3.reshape(n), conf3.reshape(n))
    stats = jnp.sum(partials, axis=0)
    cnt_c = jnp.concatenate([stats[0], stats[1]]).reshape(_LANES, 2)
    csum_c = jnp.concatenate([stats[2], stats[3]]).reshape(_LANES, 2)
    cnt = cnt_c[:_N_BINS, 0] + cnt_c[:_N_BINS, 1]
    asum = cnt_c[:_N_BINS, 1]
    csum = csum_c[:_N_BINS, 0] + csum_c[:_N_BINS, 1]
    nf = jnp.float32(n)
    safe = jnp.maximum(cnt, 1.0)
    contrib = jnp.abs(csum / safe - asum / safe) * (cnt / nf)
    ece = jnp.sum(jnp.where(cnt > 0, contrib, 0.0), keepdims=True)
    acc = jnp.sum(cnt_c[:, 1]) / nf
    return ece, acc


# speedup vs baseline: 3.8261x; 1.1056x over previous
import functools

import jax
import jax.numpy as jnp
from jax import lax
from jax.experimental import pallas as pl
from jax.experimental.pallas import tpu as pltpu
from jax.experimental.pallas import tpu_sc as plsc

_N_BINS = 10
_ROWS = 8192
_LANES = 16
_NTILES = 32
_NCELLS = 32
_CHUNK = 16384


def _dense_body(bounds_ref, logits_ref, labels_ref, conf_ref, idx_ref):
    x = logits_ref[...]
    rows, ncls = x.shape
    xt = x.T
    m = jnp.max(xt, axis=0)
    s = jnp.sum(jnp.exp(xt - m[None, :]), axis=0)
    conf = 1.0 / s
    row = lax.broadcasted_iota(jnp.int32, (ncls, rows), 0)
    pred = jnp.min(jnp.where(xt == m[None, :], row, ncls), axis=0)
    lab = labels_ref[0, 0, :]
    acci = (pred == lab).astype(jnp.int32)
    binv = jnp.zeros((rows,), jnp.int32)
    for j in range(1, _N_BINS):
        binv = binv + (conf > bounds_ref[j]).astype(jnp.int32)
    conf_ref[0, 0, :] = conf
    idx_ref[0, 0, :] = binv * 2 + acci


def _dense_stage(bounds, logits, labels3):
    n, ncls = logits.shape
    nb = n // _ROWS
    blk3 = pl.BlockSpec((1, 1, _ROWS), lambda i: (i, 0, 0))
    return pl.pallas_call(
        _dense_body,
        grid=(nb,),
        in_specs=[
            pl.BlockSpec(memory_space=pltpu.SMEM),
            pl.BlockSpec((_ROWS, ncls), lambda i: (i, 0)),
            blk3,
        ],
        out_specs=[blk3, blk3],
        out_shape=[
            jax.ShapeDtypeStruct((nb, 1, _ROWS), jnp.float32),
            jax.ShapeDtypeStruct((nb, 1, _ROWS), jnp.int32),
        ],
    )(bounds, logits, labels3)


def _make_hist_kernel(n):
    per_tile = n // _NTILES
    n_chunks = per_tile // _CHUNK
    nacc = _LANES * _NCELLS
    mesh = plsc.VectorSubcoreMesh(core_axis_name="c", subcore_axis_name="s")

    @functools.partial(
        pl.kernel,
        out_type=jax.ShapeDtypeStruct((_NTILES, 4, _LANES), jnp.float32),
        mesh=mesh,
        compiler_params=pltpu.CompilerParams(needs_layout_passes=False),
        scratch_types=[
            pltpu.VMEM((_CHUNK,), jnp.int32),
            pltpu.VMEM((_CHUNK,), jnp.float32),
            pltpu.VMEM((nacc,), jnp.float32),
            pltpu.VMEM((nacc,), jnp.float32),
            pltpu.VMEM((4, _LANES), jnp.float32),
        ],
    )
    def hist(idx_hbm, conf_hbm, out_hbm,
             idx_v, conf_v, cnt_a, csum_a, res_v):
        cid = lax.axis_index("c")
        sid = lax.axis_index("s")
        wid = sid * 2 + cid
        zero16 = jnp.zeros((_LANES,), jnp.float32)
        for r in range(nacc // _LANES):
            sl = pl.ds(r * _LANES, _LANES)
            cnt_a[sl] = zero16
            csum_a[sl] = zero16
        lanes = lax.iota(jnp.int32, _LANES)
        lane_off = lanes * _NCELLS
        ones = jnp.ones((_LANES,), jnp.float32)
        base0 = wid * per_tile
        for c in range(n_chunks):
            base = base0 + c * _CHUNK
            pltpu.sync_copy(idx_hbm.at[pl.ds(base, _CHUNK)], idx_v)
            pltpu.sync_copy(conf_hbm.at[pl.ds(base, _CHUNK)], conf_v)

            def body(i, _):
                b = idx_v[pl.ds(i * _LANES, _LANES)]
                v = conf_v[pl.ds(i * _LANES, _LANES)]
                addr = lane_off + b
                plsc.addupdate_scatter(cnt_a, [addr], ones)
                plsc.addupdate_scatter(csum_a, [addr], v)
                return 0

            lax.fori_loop(0, _CHUNK // _LANES, body, 0)
        tot = [zero16, zero16, zero16, zero16]
        for r in range(_LANES):
            for h in range(2):
                sl = pl.ds(r * _NCELLS + h * _LANES, _LANES)
                tot[h] = tot[h] + cnt_a[sl]
                tot[2 + h] = tot[2 + h] + csum_a[sl]
        for k in range(4):
            res_v[k] = tot[k]
        pltpu.sync_copy(res_v, out_hbm.at[wid])

    return hist


def kernel(logits, labels):
    n, _ = logits.shape
    bounds = jnp.linspace(0.0, 1.0, _N_BINS + 1).astype(jnp.float32)
    labels3 = labels.reshape(n // _ROWS, 1, _ROWS)
    conf3, idx3 = _dense_stage(bounds, logits, labels3)
    partials = _make_hist_kernel(n)(idx3.reshape(n), conf3.reshape(n))
    stats = jnp.sum(partials, axis=0)
    cnt_c = jnp.concatenate([stats[0], stats[1]]).reshape(_LANES, 2)
    csum_c = jnp.concatenate([stats[2], stats[3]]).reshape(_LANES, 2)
    cnt = cnt_c[:_N_BINS, 0] + cnt_c[:_N_BINS, 1]
    asum = cnt_c[:_N_BINS, 1]
    csum = csum_c[:_N_BINS, 0] + csum_c[:_N_BINS, 1]
    nf = jnp.float32(n)
    safe = jnp.maximum(cnt, 1.0)
    contrib = jnp.abs(csum / safe - asum / safe) * (cnt / nf)
    ece = jnp.sum(jnp.where(cnt > 0, contrib, 0.0), keepdims=True)
    acc = jnp.sum(cnt_c[:, 1]) / nf
    return ece, acc


# speedup vs baseline: 4.0191x; 1.0505x over previous
import functools

import jax
import jax.numpy as jnp
from jax import lax
from jax.experimental import pallas as pl
from jax.experimental.pallas import tpu as pltpu
from jax.experimental.pallas import tpu_sc as plsc

_N_BINS = 10
_ROWS = 16384
_LANES = 16
_NTILES = 32
_NCELLS = 32
_CHUNK = 16384


def _dense_body(bounds_ref, logits_ref, labels_ref, conf_ref, idx_ref):
    x = logits_ref[...]
    rows, ncls = x.shape
    xt = x.T
    m = jnp.max(xt, axis=0)
    s = jnp.sum(jnp.exp(xt - m[None, :]), axis=0)
    conf = 1.0 / s
    row = lax.broadcasted_iota(jnp.int32, (ncls, rows), 0)
    pred = jnp.min(jnp.where(xt == m[None, :], row, ncls), axis=0)
    lab = labels_ref[0, 0, :]
    acci = (pred == lab).astype(jnp.int32)
    binv = jnp.zeros((rows,), jnp.int32)
    for j in range(1, _N_BINS):
        binv = binv + (conf > bounds_ref[j]).astype(jnp.int32)
    conf_ref[0, 0, :] = conf
    idx_ref[0, 0, :] = binv * 2 + acci


def _dense_stage(bounds, logits, labels3):
    n, ncls = logits.shape
    nb = n // _ROWS
    blk3 = pl.BlockSpec((1, 1, _ROWS), lambda i: (i, 0, 0))
    return pl.pallas_call(
        _dense_body,
        grid=(nb,),
        in_specs=[
            pl.BlockSpec(memory_space=pltpu.SMEM),
            pl.BlockSpec((_ROWS, ncls), lambda i: (i, 0)),
            blk3,
        ],
        out_specs=[blk3, blk3],
        out_shape=[
            jax.ShapeDtypeStruct((nb, 1, _ROWS), jnp.float32),
            jax.ShapeDtypeStruct((nb, 1, _ROWS), jnp.int32),
        ],
    )(bounds, logits, labels3)


def _make_hist_kernel(n):
    per_tile = n // _NTILES
    n_chunks = per_tile // _CHUNK
    nacc = _LANES * _NCELLS
    mesh = plsc.VectorSubcoreMesh(core_axis_name="c", subcore_axis_name="s")

    @functools.partial(
        pl.kernel,
        out_type=jax.ShapeDtypeStruct((_NTILES, 4, _LANES), jnp.float32),
        mesh=mesh,
        compiler_params=pltpu.CompilerParams(needs_layout_passes=False),
        scratch_types=[
            pltpu.VMEM((_CHUNK,), jnp.int32),
            pltpu.VMEM((_CHUNK,), jnp.float32),
            pltpu.VMEM((nacc,), jnp.float32),
            pltpu.VMEM((nacc,), jnp.float32),
            pltpu.VMEM((4, _LANES), jnp.float32),
        ],
    )
    def hist(idx_hbm, conf_hbm, out_hbm,
             idx_v, conf_v, cnt_a, csum_a, res_v):
        cid = lax.axis_index("c")
        sid = lax.axis_index("s")
        wid = sid * 2 + cid
        zero16 = jnp.zeros((_LANES,), jnp.float32)
        for r in range(nacc // _LANES):
            sl = pl.ds(r * _LANES, _LANES)
            cnt_a[sl] = zero16
            csum_a[sl] = zero16
        lanes = lax.iota(jnp.int32, _LANES)
        lane_off = lanes * _NCELLS
        ones = jnp.ones((_LANES,), jnp.float32)
        base0 = wid * per_tile
        for c in range(n_chunks):
            base = base0 + c * _CHUNK
            pltpu.sync_copy(idx_hbm.at[pl.ds(base, _CHUNK)], idx_v)
            pltpu.sync_copy(conf_hbm.at[pl.ds(base, _CHUNK)], conf_v)

            def body(i, _):
                b = idx_v[pl.ds(i * _LANES, _LANES)]
                v = conf_v[pl.ds(i * _LANES, _LANES)]
                addr = lane_off + b
                plsc.addupdate_scatter(cnt_a, [addr], ones)
                plsc.addupdate_scatter(csum_a, [addr], v)
                return 0

            lax.fori_loop(0, _CHUNK // _LANES, body, 0)
        tot = [zero16, zero16, zero16, zero16]
        for r in range(_LANES):
            for h in range(2):
                sl = pl.ds(r * _NCELLS + h * _LANES, _LANES)
                tot[h] = tot[h] + cnt_a[sl]
                tot[2 + h] = tot[2 + h] + csum_a[sl]
        for k in range(4):
            res_v[k] = tot[k]
        pltpu.sync_copy(res_v, out_hbm.at[wid])

    return hist


def kernel(logits, labels):
    n, _ = logits.shape
    bounds = jnp.linspace(0.0, 1.0, _N_BINS + 1).astype(jnp.float32)
    labels3 = labels.reshape(n // _ROWS, 1, _ROWS)
    conf3, idx3 = _dense_stage(bounds, logits, labels3)
    partials = _make_hist_kernel(n)(idx3.reshape(n), conf3.reshape(n))
    stats = jnp.sum(partials, axis=0)
    cnt_c = jnp.concatenate([stats[0], stats[1]]).reshape(_LANES, 2)
    csum_c = jnp.concatenate([stats[2], stats[3]]).reshape(_LANES, 2)
    cnt = cnt_c[:_N_BINS, 0] + cnt_c[:_N_BINS, 1]
    asum = cnt_c[:_N_BINS, 1]
    csum = csum_c[:_N_BINS, 0] + csum_c[:_N_BINS, 1]
    nf = jnp.float32(n)
    safe = jnp.maximum(cnt, 1.0)
    contrib = jnp.abs(csum / safe - asum / safe) * (cnt / nf)
    ece = jnp.sum(jnp.where(cnt > 0, contrib, 0.0), keepdims=True)
    acc = jnp.sum(cnt_c[:, 1]) / nf
    return ece, acc
